# Initial kernel scaffold; baseline (speedup 1.0000x reference)
import jax
import jax.numpy as jnp
from jax.experimental import pallas as pl


def kernel(token_ids, table0, table1, table2, table3):
    raise NotImplementedError("write your pallas kernel here")


# speedup vs baseline: 11.5740x; 11.5740x over previous
import functools

import jax
import jax.numpy as jnp
from jax import lax
from jax.experimental import pallas as pl
from jax.experimental.pallas import tpu as pltpu
from jax.experimental.pallas import tpu_sc as plsc

_HASH_SEEDS = (2654435769, 3210233709, 2496678331, 3249880090)
_TBL = 8192
_D = 64
_L = 16
_NC = 2
_NS = 16
_NW = _NC * _NS

_B = 4096
_S = 50
_NB = _S - 1
_RPW = _B // _NW
_TPW = _RPW * _S
_BPW = _RPW * _NB
_CHUNK = 128
_NCHUNK = _BPW // _CHUNK
_GRP = _BPW // _L


def _gf_hash(x, seed):
    x = x ^ jnp.uint32(seed)
    x = (x ^ (x >> jnp.uint32(16))) * jnp.uint32(2146121005)
    x = (x ^ (x >> jnp.uint32(15))) * jnp.uint32(2221713035)
    x = x ^ (x >> jnp.uint32(16))
    return (x & jnp.uint32(_TBL - 1)).astype(jnp.int32)


def _body(tok_hbm, t0, t1, t2, t3, out_hbm,
          tok_v, idx0, idx1, idx2, idx3, b0, b1, b2, b3, ob, sem):
    wid = lax.axis_index("s") * _NC + lax.axis_index("c")

    pltpu.sync_copy(tok_hbm.at[pl.ds(wid * _TPW, _TPW)], tok_v)

    lane = lax.iota(jnp.int32, _L)
    idx_refs = (idx0, idx1, idx2, idx3)

    def hash_body(g, carry):
        i = g * _L + lane
        pos = i + lax.div(i, lax.full_like(i, _NB))
        left = plsc.load_gather(tok_v, [pos]).astype(jnp.uint32)
        right = plsc.load_gather(tok_v, [pos + 1]).astype(jnp.uint32)
        bg = (left << jnp.uint32(10)) | right
        r = g >> 3
        col = (g & 7) * _L
        for h in range(4):
            idx_refs[h][r, pl.ds(col, _L)] = _gf_hash(bg, _HASH_SEEDS[h])
        return carry

    lax.fori_loop(0, _GRP, hash_body, 0)

    obase = wid * _BPW
    tables = (t0, t1, t2, t3)
    bufs = (b0, b1, b2, b3)

    def chunk_body(c, carry):
        cps = [pltpu.async_copy(tables[h].at[idx_refs[h].at[c]], bufs[h], sem)
               for h in range(4)]
        for cp in cps:
            cp.wait()

        def comb(r, carry2):
            for c4 in range(_D // _L):
                s = pl.ds(c4 * _L, _L)
                ob[r, s] = ((b0[r, s] + b1[r, s]) + (b2[r, s] + b3[r, s])) \
                    * jnp.float32(0.25)
            return carry2

        lax.fori_loop(0, _CHUNK, comb, 0)
        pltpu.sync_copy(ob, out_hbm.at[pl.ds(obase + c * _CHUNK, _CHUNK)])
        return carry

    lax.fori_loop(0, _NCHUNK, chunk_body, 0)


_sc_call = functools.partial(
    pl.kernel,
    out_type=jax.ShapeDtypeStruct((_B * _NB, _D), jnp.float32),
    mesh=plsc.VectorSubcoreMesh(
        core_axis_name="c", subcore_axis_name="s",
        num_cores=_NC, num_subcores=_NS),
    scratch_types=[
        pltpu.VMEM((_TPW,), jnp.int32),
        pltpu.VMEM((_NCHUNK, _CHUNK), jnp.int32),
        pltpu.VMEM((_NCHUNK, _CHUNK), jnp.int32),
        pltpu.VMEM((_NCHUNK, _CHUNK), jnp.int32),
        pltpu.VMEM((_NCHUNK, _CHUNK), jnp.int32),
        pltpu.VMEM((_CHUNK, _D), jnp.float32),
        pltpu.VMEM((_CHUNK, _D), jnp.float32),
        pltpu.VMEM((_CHUNK, _D), jnp.float32),
        pltpu.VMEM((_CHUNK, _D), jnp.float32),
        pltpu.VMEM((_CHUNK, _D), jnp.float32),
        pltpu.SemaphoreType.DMA,
    ],
    compiler_params=pltpu.CompilerParams(
        needs_layout_passes=False, use_tc_tiling_on_sc=False),
)(_body)


@jax.jit
def kernel(token_ids, table0, table1, table2, table3):
    out = _sc_call(token_ids.reshape(-1), table0, table1, table2, table3)
    return out.reshape(_B, _NB, _D)


# speedup vs baseline: 15.2619x; 1.3186x over previous
import functools

import jax
import jax.numpy as jnp
from jax import lax
from jax.experimental import pallas as pl
from jax.experimental.pallas import tpu as pltpu
from jax.experimental.pallas import tpu_sc as plsc

_HASH_SEEDS = (2654435769, 3210233709, 2496678331, 3249880090)
_TBL = 8192
_D = 64
_L = 16
_NC = 2
_NS = 16
_NW = _NC * _NS

_B = 4096
_S = 50
_NB = _S - 1
_RPW = _B // _NW
_TPW = _RPW * _S
_BPW = _RPW * _NB
_CHUNK = 128
_NCHUNK = _BPW // _CHUNK
_GRP = _BPW // _L


def _gf_hash(x, seed):
    x = x ^ jnp.uint32(seed)
    x = (x ^ (x >> jnp.uint32(16))) * jnp.uint32(2146121005)
    x = (x ^ (x >> jnp.uint32(15))) * jnp.uint32(2221713035)
    x = x ^ (x >> jnp.uint32(16))
    return (x & jnp.uint32(_TBL - 1)).astype(jnp.int32)


def _body(tok_hbm, t0, t1, t2, t3, out_hbm,
          tok_v, idx0, idx1, idx2, idx3,
          a0, a1, a2, a3, b0, b1, b2, b3, oa, ob,
          sga, sgb, soa, sob):
    wid = lax.axis_index("s") * _NC + lax.axis_index("c")

    pltpu.sync_copy(tok_hbm.at[pl.ds(wid * _TPW, _TPW)], tok_v)

    lane = lax.iota(jnp.int32, _L)
    idx_refs = (idx0, idx1, idx2, idx3)
    tables = (t0, t1, t2, t3)

    @plsc.parallel_loop(0, _GRP, unroll=2)
    def _hash_loop(g):
        i = g * _L + lane
        pos = i + lax.div(i, lax.full_like(i, _NB))
        left = plsc.load_gather(tok_v, [pos]).astype(jnp.uint32)
        right = plsc.load_gather(tok_v, [pos + 1]).astype(jnp.uint32)
        bg = (left << jnp.uint32(10)) | right
        r = g >> 3
        col = (g & 7) * _L
        for h in range(4):
            idx_refs[h][r, pl.ds(col, _L)] = _gf_hash(bg, _HASH_SEEDS[h])

    obase = wid * _BPW
    set_a = ((a0, a1, a2, a3), oa, sga, soa)
    set_b = ((b0, b1, b2, b3), ob, sgb, sob)

    def issue(c, bset):
        bufs, _, sg, _ = bset
        for h in range(4):
            pltpu.async_copy(tables[h].at[idx_refs[h].at[c]], bufs[h], sg)

    def wait_gathers(bset):
        bufs, _, sg, _ = bset
        for h in range(4):
            pltpu.make_async_copy(tables[h].at[idx_refs[h].at[0]],
                                  bufs[h], sg).wait()

    def wait_out(bset):
        _, o, _, so = bset
        pltpu.make_async_copy(o, out_hbm.at[pl.ds(obase, _CHUNK)], so).wait()

    def combine(bset):
        bufs, o, _, _ = bset
        c0, c1, c2, c3 = bufs

        @plsc.parallel_loop(0, _CHUNK, unroll=4)
        def _comb(r):
            for c4 in range(_D // _L):
                s = pl.ds(c4 * _L, _L)
                o[r, s] = ((c0[r, s] + c1[r, s]) + (c2[r, s] + c3[r, s])) \
                    * jnp.float32(0.25)

    def out_dma(c, bset):
        _, o, _, so = bset
        pltpu.async_copy(o, out_hbm.at[pl.ds(obase + c * _CHUNK, _CHUNK)], so)

    issue(0, set_a)

    def pair_body(k, carry):
        ca = 2 * k
        cb = ca + 1
        issue(cb, set_b)
        wait_gathers(set_a)

        @pl.when(k > 0)
        def _():
            wait_out(set_a)

        combine(set_a)
        out_dma(ca, set_a)
        issue(ca + 2, set_a)
        wait_gathers(set_b)

        @pl.when(k > 0)
        def _():
            wait_out(set_b)

        combine(set_b)
        out_dma(cb, set_b)
        return carry

    lax.fori_loop(0, (_NCHUNK - 1) // 2, pair_body, 0)

    wait_gathers(set_a)
    wait_out(set_a)
    combine(set_a)
    out_dma(_NCHUNK - 1, set_a)
    wait_out(set_a)
    wait_out(set_b)


_sc_call = functools.partial(
    pl.kernel,
    out_type=jax.ShapeDtypeStruct((_B * _NB, _D), jnp.float32),
    mesh=plsc.VectorSubcoreMesh(
        core_axis_name="c", subcore_axis_name="s",
        num_cores=_NC, num_subcores=_NS),
    scratch_types=[
        pltpu.VMEM((_TPW,), jnp.int32),
        pltpu.VMEM((_NCHUNK, _CHUNK), jnp.int32),
        pltpu.VMEM((_NCHUNK, _CHUNK), jnp.int32),
        pltpu.VMEM((_NCHUNK, _CHUNK), jnp.int32),
        pltpu.VMEM((_NCHUNK, _CHUNK), jnp.int32),
        pltpu.VMEM((_CHUNK, _D), jnp.float32),
        pltpu.VMEM((_CHUNK, _D), jnp.float32),
        pltpu.VMEM((_CHUNK, _D), jnp.float32),
        pltpu.VMEM((_CHUNK, _D), jnp.float32),
        pltpu.VMEM((_CHUNK, _D), jnp.float32),
        pltpu.VMEM((_CHUNK, _D), jnp.float32),
        pltpu.VMEM((_CHUNK, _D), jnp.float32),
        pltpu.VMEM((_CHUNK, _D), jnp.float32),
        pltpu.VMEM((_CHUNK, _D), jnp.float32),
        pltpu.VMEM((_CHUNK, _D), jnp.float32),
        pltpu.SemaphoreType.DMA,
        pltpu.SemaphoreType.DMA,
        pltpu.SemaphoreType.DMA,
        pltpu.SemaphoreType.DMA,
    ],
    compiler_params=pltpu.CompilerParams(
        needs_layout_passes=False, use_tc_tiling_on_sc=False),
)(_body)


@jax.jit
def kernel(token_ids, table0, table1, table2, table3):
    out = _sc_call(token_ids.reshape(-1), table0, table1, table2, table3)
    return out.reshape(_B, _NB, _D)
